# baseline (speedup 1.0000x reference)
import functools

import jax
import jax.numpy as jnp
from jax import lax
from jax.experimental import pallas as pl
from jax.experimental.pallas import tpu as pltpu
from jax.experimental.pallas import tpu_sc as plsc

B = 16384
NC = 2
NS = 16
L = 16
NW = NC * NS
BPW = B // NW
NG = BPW // L

H_V, H_D = 24, 8
D_V, D_D = 7, 4
M_V, M_D = 31, 6
OUT_D = H_D + D_D + M_D


def _build_kernel():
    mesh = plsc.VectorSubcoreMesh(core_axis_name="c", subcore_axis_name="s")

    @functools.partial(
        pl.kernel,
        mesh=mesh,
        compiler_params=pltpu.CompilerParams(needs_layout_passes=False),
        out_type=jax.ShapeDtypeStruct((B * OUT_D,), jnp.float32),
        scratch_types=[
            pltpu.VMEM((H_V * H_D,), jnp.float32),
            pltpu.VMEM((D_V * D_D,), jnp.float32),
            pltpu.VMEM((M_V * M_D,), jnp.float32),
            pltpu.VMEM((BPW,), jnp.int32),
            pltpu.VMEM((BPW,), jnp.int32),
            pltpu.VMEM((BPW,), jnp.int32),
            pltpu.VMEM((BPW * OUT_D,), jnp.float32),
        ],
    )
    def emb(hour_hbm, dow_hbm, dom_hbm, ht_hbm, dt_hbm, mt_hbm, out_hbm,
            ht_v, dt_v, mt_v, hi_v, di_v, mi_v, out_v):
        wid = lax.axis_index("s") * NC + lax.axis_index("c")
        base = wid * BPW

        pltpu.sync_copy(ht_hbm, ht_v)
        pltpu.sync_copy(dt_hbm, dt_v)
        pltpu.sync_copy(mt_hbm, mt_v)
        pltpu.sync_copy(hour_hbm.at[pl.ds(base, BPW)], hi_v)
        pltpu.sync_copy(dow_hbm.at[pl.ds(base, BPW)], di_v)
        pltpu.sync_copy(dom_hbm.at[pl.ds(base, BPW)], mi_v)

        lanes = lax.iota(jnp.int32, L)

        for g in range(NG):
            rows18 = (g * L * OUT_D) + lanes * OUT_D
            hi = hi_v[pl.ds(g * L, L)] * H_D
            di = di_v[pl.ds(g * L, L)] * D_D
            mi = mi_v[pl.ds(g * L, L)] * M_D
            for j in range(H_D):
                v = plsc.load_gather(ht_v, [hi + j])
                plsc.store_scatter(out_v, [rows18 + j], v)
            for j in range(D_D):
                v = plsc.load_gather(dt_v, [di + j])
                plsc.store_scatter(out_v, [rows18 + (H_D + j)], v)
            for j in range(M_D):
                v = plsc.load_gather(mt_v, [mi + j])
                plsc.store_scatter(out_v, [rows18 + (H_D + D_D + j)], v)

        pltpu.sync_copy(out_v, out_hbm.at[pl.ds(base * OUT_D, BPW * OUT_D)])

    return emb


_EMB = _build_kernel()


@jax.jit
def kernel(hour, dow, dom, hour_table, dow_table, dom_table):
    flat = _EMB(hour.astype(jnp.int32), dow.astype(jnp.int32),
                dom.astype(jnp.int32), hour_table.reshape(-1),
                dow_table.reshape(-1), dom_table.reshape(-1))
    return flat.reshape(B, OUT_D)


# speedup vs baseline: 1.0677x; 1.0677x over previous
import functools

import jax
import jax.numpy as jnp
from jax import lax
from jax.experimental import pallas as pl
from jax.experimental.pallas import tpu as pltpu
from jax.experimental.pallas import tpu_sc as plsc

B = 16384
NC = 2
NS = 16
L = 16
NW = NC * NS
BPW = B // NW
NG = BPW // L

H_V, H_D = 24, 8
D_V, D_D = 7, 4
M_V, M_D = 31, 6
OUT_D = H_D + D_D + M_D


def _build_kernel():
    mesh = plsc.VectorSubcoreMesh(core_axis_name="c", subcore_axis_name="s")

    @functools.partial(
        pl.kernel,
        mesh=mesh,
        compiler_params=pltpu.CompilerParams(needs_layout_passes=False),
        out_type=jax.ShapeDtypeStruct((B, OUT_D), jnp.float32),
        scratch_types=[
            pltpu.VMEM((H_V, H_D), jnp.float32),
            pltpu.VMEM((D_V, D_D), jnp.float32),
            pltpu.VMEM((M_V, M_D), jnp.float32),
            pltpu.VMEM((BPW,), jnp.int32),
            pltpu.VMEM((BPW,), jnp.int32),
            pltpu.VMEM((BPW,), jnp.int32),
            pltpu.VMEM((BPW, OUT_D), jnp.float32),
        ],
    )
    def emb(hour_hbm, dow_hbm, dom_hbm, ht_hbm, dt_hbm, mt_hbm, out_hbm,
            ht_v, dt_v, mt_v, hi_v, di_v, mi_v, out_v):
        wid = lax.axis_index("s") * NC + lax.axis_index("c")
        base = wid * BPW

        pltpu.sync_copy(ht_hbm, ht_v)
        pltpu.sync_copy(dt_hbm, dt_v)
        pltpu.sync_copy(mt_hbm, mt_v)
        pltpu.sync_copy(hour_hbm.at[pl.ds(base, BPW)], hi_v)
        pltpu.sync_copy(dow_hbm.at[pl.ds(base, BPW)], di_v)
        pltpu.sync_copy(dom_hbm.at[pl.ds(base, BPW)], mi_v)

        lanes = lax.iota(jnp.int32, L)

        def body(g, carry):
            rows = g * L + lanes
            hi = hi_v[pl.ds(g * L, L)]
            di = di_v[pl.ds(g * L, L)]
            mi = mi_v[pl.ds(g * L, L)]
            for j in range(H_D):
                col = jnp.full((L,), j, jnp.int32)
                v = plsc.load_gather(ht_v, [hi, col])
                plsc.store_scatter(out_v, [rows, col], v)
            for j in range(D_D):
                col = jnp.full((L,), j, jnp.int32)
                ocol = jnp.full((L,), H_D + j, jnp.int32)
                v = plsc.load_gather(dt_v, [di, col])
                plsc.store_scatter(out_v, [rows, ocol], v)
            for j in range(M_D):
                col = jnp.full((L,), j, jnp.int32)
                ocol = jnp.full((L,), H_D + D_D + j, jnp.int32)
                v = plsc.load_gather(mt_v, [mi, col])
                plsc.store_scatter(out_v, [rows, ocol], v)
            return carry

        lax.fori_loop(0, NG, body, 0)

        pltpu.sync_copy(out_v, out_hbm.at[pl.ds(base, BPW)])

    return emb


_EMB = _build_kernel()


@jax.jit
def kernel(hour, dow, dom, hour_table, dow_table, dom_table):
    return _EMB(hour.astype(jnp.int32), dow.astype(jnp.int32),
                dom.astype(jnp.int32), hour_table, dow_table, dom_table)


# speedup vs baseline: 1.8234x; 1.7078x over previous
import functools

import jax
import jax.numpy as jnp
from jax import lax
from jax.experimental import pallas as pl
from jax.experimental.pallas import tpu as pltpu
from jax.experimental.pallas import tpu_sc as plsc

B = 16384
NC = 2
NS = 16
L = 16
NW = NC * NS
BPW = B // NW
NG = BPW // L

H_V, H_D = 24, 8
D_V, D_D = 7, 4
M_V, M_D = 31, 6
OUT_D = H_D + D_D + M_D


def _build_kernel():
    mesh = plsc.VectorSubcoreMesh(core_axis_name="c", subcore_axis_name="s")

    @functools.partial(
        pl.kernel,
        mesh=mesh,
        compiler_params=pltpu.CompilerParams(needs_layout_passes=False),
        out_type=jax.ShapeDtypeStruct((OUT_D, B), jnp.float32),
        scratch_types=[
            pltpu.VMEM((H_D, H_V), jnp.float32),
            pltpu.VMEM((D_D, D_V), jnp.float32),
            pltpu.VMEM((M_D, M_V), jnp.float32),
            pltpu.VMEM((BPW,), jnp.int32),
            pltpu.VMEM((BPW,), jnp.int32),
            pltpu.VMEM((BPW,), jnp.int32),
            pltpu.VMEM((OUT_D, BPW), jnp.float32),
        ],
    )
    def emb(hour_hbm, dow_hbm, dom_hbm, ht_hbm, dt_hbm, mt_hbm, out_hbm,
            ht_v, dt_v, mt_v, hi_v, di_v, mi_v, out_v):
        wid = lax.axis_index("s") * NC + lax.axis_index("c")
        base = wid * BPW

        pltpu.sync_copy(ht_hbm, ht_v)
        pltpu.sync_copy(dt_hbm, dt_v)
        pltpu.sync_copy(mt_hbm, mt_v)
        pltpu.sync_copy(hour_hbm.at[pl.ds(base, BPW)], hi_v)
        pltpu.sync_copy(dow_hbm.at[pl.ds(base, BPW)], di_v)
        pltpu.sync_copy(dom_hbm.at[pl.ds(base, BPW)], mi_v)

        def body(g, carry):
            sl = pl.ds(g * L, L)
            hi = hi_v[sl]
            di = di_v[sl]
            mi = mi_v[sl]
            for j in range(H_D):
                col = jnp.full((L,), j, jnp.int32)
                out_v[j, sl] = plsc.load_gather(ht_v, [col, hi])
            for j in range(D_D):
                col = jnp.full((L,), j, jnp.int32)
                out_v[H_D + j, sl] = plsc.load_gather(dt_v, [col, di])
            for j in range(M_D):
                col = jnp.full((L,), j, jnp.int32)
                out_v[H_D + D_D + j, sl] = plsc.load_gather(mt_v, [col, mi])
            return carry

        lax.fori_loop(0, NG, body, 0)

        pltpu.sync_copy(out_v, out_hbm.at[:, pl.ds(base, BPW)])

    return emb


_EMB = _build_kernel()


@jax.jit
def kernel(hour, dow, dom, hour_table, dow_table, dom_table):
    out_t = _EMB(hour.astype(jnp.int32), dow.astype(jnp.int32),
                 dom.astype(jnp.int32), hour_table.T, dow_table.T,
                 dom_table.T)
    return out_t.T
